# baseline (speedup 1.0000x reference)
import jax
import jax.numpy as jnp
from jax import lax
from jax.experimental import pallas as pl
from jax.experimental.pallas import tpu as pltpu
from jax.experimental.pallas import tpu_sc as plsc

NC, NS, LANES = 2, 16, 16
NW = NC * NS

B, H, D = 4096, 50, 128
TOTAL = B * H
B_PER_W = TOTAL // NW
CHUNK = 64
NCHUNK = B_PER_W // CHUNK
NB = 10
LOOKAHEAD = 6


def _body(idx_hbm, table_hbm, out_hbm, idx_v, rows, gsem, ssem):
    wid = lax.axis_index("s") * NC + lax.axis_index("c")
    base = wid * B_PER_W
    pltpu.sync_copy(idx_hbm.at[pl.ds(base, B_PER_W)], idx_v)

    def gather(chunk, b):
        return pltpu.make_async_copy(
            table_hbm.at[idx_v.at[pl.ds(chunk * CHUNK, CHUNK)]],
            rows.at[b],
            gsem.at[b],
        )

    def store(chunk, b):
        return pltpu.make_async_copy(
            rows.at[b],
            out_hbm.at[pl.ds(base + chunk * CHUNK, CHUNK)],
            ssem.at[b],
        )

    for b in range(LOOKAHEAD):
        gather(b, b).start()

    @pl.loop(0, NCHUNK, step=NB)
    def _outer(v0):
        for db in range(NB):
            v = v0 + db
            bn = (db + LOOKAHEAD) % NB

            @pl.when(v + LOOKAHEAD < NCHUNK)
            def _():
                @pl.when(v >= NB - LOOKAHEAD)
                def _():
                    store(v - (NB - LOOKAHEAD), bn).wait()

                gather(v + LOOKAHEAD, bn).start()

            zacc = jnp.zeros((LANES,), jnp.int32)
            for g in range(CHUNK // LANES):
                ivals = idx_v[pl.ds(v * CHUNK + g * LANES, LANES)]
                zacc = zacc | (ivals == 0).astype(jnp.int32)
            any_pad = jnp.max(zacc)

            gather(v, db).wait()

            @pl.when(any_pad > 0)
            def _():
                for g in range(CHUNK // LANES):
                    ivals = idx_v[pl.ds(v * CHUNK + g * LANES, LANES)]
                    zm = ivals == 0

                    @pl.when(jnp.max(zm.astype(jnp.int32)) > 0)
                    def _():
                        rowids = g * LANES + lax.iota(jnp.int32, LANES)
                        zeros = jnp.zeros((LANES,), jnp.float32)

                        @pl.loop(0, D)
                        def _col(c):
                            colv = jnp.full((LANES,), c, jnp.int32)
                            plsc.store_scatter(rows.at[db], [rowids, colv], zeros, mask=zm)

            store(v, db).start()

    for db in range(NB):
        store(NCHUNK - NB + db, db).wait()


def kernel(idx, embeddings, padding_mult):
    del padding_mult
    idx_flat = idx.T.reshape(-1)
    mesh = plsc.VectorSubcoreMesh(core_axis_name="c", subcore_axis_name="s")
    k = pl.kernel(
        _body,
        out_type=jax.ShapeDtypeStruct((TOTAL, D), jnp.float32),
        mesh=mesh,
        compiler_params=pltpu.CompilerParams(needs_layout_passes=False),
        scratch_types=[
            pltpu.VMEM((B_PER_W,), jnp.int32),
            pltpu.VMEM((NB, CHUNK, D), jnp.float32),
            pltpu.SemaphoreType.DMA((NB,)),
            pltpu.SemaphoreType.DMA((NB,)),
        ],
    )
    out = k(idx_flat, embeddings)
    return out.reshape(H, B, D).transpose(1, 0, 2)
